# baseline (speedup 1.0000x reference)
import jax
import jax.numpy as jnp
from jax import lax
from jax.experimental import pallas as pl
from jax.experimental.pallas import tpu as pltpu
from jax.experimental.pallas import tpu_sc as plsc

_INFO = plsc.get_sparse_core_info()
_NC = _INFO.num_cores
_NS = _INFO.num_subcores
_NW = _NC * _NS

_B, _C, _H, _W = 4, 3, 512, 512
_PLANE = _H * _W
_PIX = _B * _PLANE
_PPW = _PIX // _NW
_CH = 2048
_NCHUNK = _PPW // _CH
_G = 2048
_NG = _CH // _G
_NVEC = _CH // 16


def _worker_base():
    wid = lax.axis_index("s") * _NC + lax.axis_index("c")
    bi = wid // 8
    po = (wid % 8) * _PPW
    return bi * (_C * _PLANE) + po


def _pipeline(img_hbm, extra_in, cols, res_hbm, xbufs2, idxbuf2, gbufs2,
              obufs2, gsem, isem, osem, fx):
    base = _worker_base()

    def stage_in(s, p):
        off = base + s * _CH
        hs = [pltpu.async_copy(img_hbm.at[pl.ds(off + c * _PLANE, _CH)],
                               xbufs2[p][c], isem)
              for c in range(3)]
        for hbm, bufs2 in extra_in:
            hs += [pltpu.async_copy(hbm.at[pl.ds(off + c * _PLANE, _CH)],
                                    bufs2[p][c], isem)
                   for c in range(3)]
        return hs

    def do_idx(p):
        xb, ib = xbufs2[p], idxbuf2[p]

        def idx_body(j, _):
            q = pl.ds(j * 16, 16)
            fidx = xb[0][q] * 65536.0 + xb[1][q] * 256.0 + xb[2][q]
            ib[q] = fidx.astype(jnp.int32)
            return 0

        lax.fori_loop(0, _NVEC, idx_body, 0, unroll=4)

    def fire(p):
        ib, gb = idxbuf2[p], gbufs2[p]
        hs = []
        for g in range(_NG):
            gs = pl.ds(g * _G, _G)
            isl = ib.at[gs]
            for c in range(3):
                hs.append(pltpu.async_copy(cols[c].at[isl],
                                           gb[c].at[gs], gsem))
        return hs

    def finish(s, p):
        ob = obufs2[p]

        def fx_body(j, _):
            q = pl.ds(j * 16, 16)
            res = fx(q, p)
            for c in range(3):
                ob[c][q] = res[c]
            return 0

        lax.fori_loop(0, _NVEC, fx_body, 0, unroll=4)
        off = base + s * _CH
        return [pltpu.async_copy(ob[c],
                                 res_hbm.at[pl.ds(off + c * _PLANE, _CH)],
                                 osem)
                for c in range(3)]

    h_img = {0: stage_in(0, 0), 1: None}
    h_gat = {0: None, 1: None}
    h_out = {0: None, 1: None}
    prev = None
    for s in range(_NCHUNK):
        cur = s % 2
        for h in h_img[cur]:
            h.wait()
        do_idx(cur)
        h_gat[cur] = fire(cur)
        if prev is not None:
            pv = prev % 2
            for h in h_gat[pv]:
                h.wait()
            if h_out[pv] is not None:
                for h in h_out[pv]:
                    h.wait()
            h_out[pv] = finish(prev, pv)
        if s + 1 < _NCHUNK:
            h_img[(s + 1) % 2] = stage_in(s + 1, (s + 1) % 2)
        prev = s
    pv = prev % 2
    for h in h_gat[pv]:
        h.wait()
    if h_out[pv] is not None:
        for h in h_out[pv]:
            h.wait()
    for h in finish(prev, pv):
        h.wait()


def _k1_body(img_hbm, b0_hbm, b1_hbm, b2_hbm, p1_hbm,
             xbufs2, idxbuf2, bbufs2, obufs2, gsem, isem, osem):

    def fx(q, p):
        bb = bbufs2[p]
        return tuple((bb[c][q] + 1.0) * 127.0 for c in range(3))

    _pipeline(img_hbm, [], (b0_hbm, b1_hbm, b2_hbm), p1_hbm,
              xbufs2, idxbuf2, bbufs2, obufs2, gsem, isem, osem, fx)


def _k2_body(img_hbm, p1_hbm, w0_hbm, w1_hbm, w2_hbm, out_hbm,
             xbufs2, pbufs2, idxbuf2, wbufs2, obufs2, gsem, isem, osem):

    def fx(q, p):
        xb, pb, wb = xbufs2[p], pbufs2[p], wbufs2[p]
        return tuple(wb[c][q] * (xb[c][q] - 127.0) + pb[c][q]
                     for c in range(3))

    _pipeline(img_hbm, [(p1_hbm, pbufs2)], (w0_hbm, w1_hbm, w2_hbm), out_hbm,
              xbufs2, idxbuf2, wbufs2, obufs2, gsem, isem, osem, fx)


@jax.jit
def kernel(img, w, b):
    img_flat = img.reshape(-1)
    mesh = plsc.VectorSubcoreMesh(core_axis_name="c", subcore_axis_name="s")
    fbuf = pltpu.VMEM((_CH,), jnp.float32)
    ibuf = pltpu.VMEM((_CH,), jnp.int32)
    p1 = pl.kernel(
        _k1_body,
        out_type=jax.ShapeDtypeStruct((_B * _C * _PLANE,), jnp.float32),
        mesh=mesh,
        scratch_types=[
            [[fbuf] * 3] * 2,
            [ibuf] * 2,
            [[fbuf] * 3] * 2,
            [[fbuf] * 3] * 2,
            pltpu.SemaphoreType.DMA,
            pltpu.SemaphoreType.DMA,
            pltpu.SemaphoreType.DMA,
        ],
    )(img_flat, b[:, 0], b[:, 1], b[:, 2])
    out_flat = pl.kernel(
        _k2_body,
        out_type=jax.ShapeDtypeStruct((_B * _C * _PLANE,), jnp.float32),
        mesh=mesh,
        scratch_types=[
            [[fbuf] * 3] * 2,
            [[fbuf] * 3] * 2,
            [ibuf] * 2,
            [[fbuf] * 3] * 2,
            [[fbuf] * 3] * 2,
            pltpu.SemaphoreType.DMA,
            pltpu.SemaphoreType.DMA,
            pltpu.SemaphoreType.DMA,
        ],
    )(img_flat, p1, w[:, 0], w[:, 1], w[:, 2])
    return out_flat.reshape(_B, _C, _H, _W)
